# baseline (speedup 1.0000x reference)
import functools

import jax
import jax.numpy as jnp
from jax import lax
from jax.experimental import pallas as pl
from jax.experimental.pallas import tpu as pltpu
from jax.experimental.pallas import tpu_sc as plsc

N = 4194304
KTOP = 419431
NC = 2
NS = 16
NW = NC * NS
L = 16

CHUNK_A = N // NW
WA = 8192
NWIN = CHUNK_A // WA
BINS_A = 4096
SHIFT_A = 20

CANDCAP = 589824
CHUNK_R = CANDCAP // NW
BINS_R = 256
CAP_T = 32768
SENT_I32 = -1


_CP = pltpu.CompilerParams(needs_layout_passes=False)


def _wid():
    return lax.axis_index("s") * NC + lax.axis_index("c")


def _key_from_f32(x):
    u = lax.bitcast_convert_type(x, jnp.uint32)
    m = u >> 31
    return u ^ (m * jnp.uint32(0x7FFFFFFF) + jnp.uint32(0x80000000))


@functools.cache
def _build():
    mesh = plsc.VectorSubcoreMesh(
        core_axis_name="c", subcore_axis_name="s", num_cores=NC, num_subcores=NS
    )

    @functools.partial(
        pl.kernel,
        out_type=jax.ShapeDtypeStruct((NW, L * BINS_A), jnp.int32),
        mesh=mesh,
        compiler_params=_CP,
        scratch_types=[
            pltpu.VMEM((WA,), jnp.float32),
            pltpu.VMEM((L * BINS_A,), jnp.int32),
        ],
    )
    def hist_kernel(w_hbm, hist_hbm, wbuf, hist_v):
        wid = _wid()
        lane = lax.iota(jnp.int32, L)
        ones = jnp.ones((L,), jnp.int32)

        def zbody(i, c):
            hist_v[pl.ds(i * L, L)] = jnp.zeros((L,), jnp.int32)
            return c

        lax.fori_loop(0, (L * BINS_A) // L, zbody, 0)

        base = wid * CHUNK_A

        def wbody(g, c):
            pltpu.sync_copy(w_hbm.at[pl.ds(pl.multiple_of(base + g * WA, 16), WA)], wbuf)

            def ibody(j, c2):
                key = _key_from_f32(wbuf[pl.ds(j * L, L)])
                b = (key >> SHIFT_A).astype(jnp.int32)
                plsc.addupdate_scatter(hist_v, [lane * BINS_A + b], ones)
                return c2

            return lax.fori_loop(0, WA // L, ibody, c)

        lax.fori_loop(0, NWIN, wbody, 0)
        pltpu.sync_copy(hist_v, hist_hbm.at[wid])

    @functools.partial(
        pl.kernel,
        out_type=(
            jax.ShapeDtypeStruct((CANDCAP,), jnp.int32),
            jax.ShapeDtypeStruct((CANDCAP,), jnp.int32),
        ),
        mesh=mesh,
        compiler_params=_CP,
        scratch_types=[
            pltpu.VMEM((WA,), jnp.float32),
            pltpu.VMEM((CAP_T + L,), jnp.int32),
            pltpu.VMEM((CAP_T + L,), jnp.int32),
            pltpu.VMEM((WA,), jnp.int32),
            pltpu.VMEM((40,), jnp.int32),
        ],
    )
    def compact_kernel(w_hbm, par_hbm, ck_hbm, ci_hbm, wbuf, kbuf, ibuf, sbuf, pbuf):
        wid = _wid()
        lane = lax.iota(jnp.int32, L)
        pltpu.sync_copy(par_hbm, pbuf)
        p0 = pbuf[pl.ds(0, L)]
        thr_v = lax.bitcast_convert_type(jnp.full((L,), 0, jnp.int32) + p0[0], jnp.uint32)
        bw_v = plsc.load_gather(pbuf, [jnp.full((L,), 0, jnp.int32) + (1 + wid)])
        base_w = bw_v[0]
        cbase = wid * CHUNK_A

        def wbody(g, off):
            pltpu.sync_copy(w_hbm.at[pl.ds(pl.multiple_of(cbase + g * WA, 16), WA)], wbuf)

            def ibody(j, off2):
                key = _key_from_f32(wbuf[pl.ds(j * L, L)])
                msk = key >= thr_v
                fkey = lax.bitcast_convert_type(~key, jnp.int32)
                gidx = cbase + g * WA + j * L + lane
                plsc.store_compressed(kbuf.at[pl.ds(off2, L)], fkey, mask=msk)
                plsc.store_compressed(ibuf.at[pl.ds(off2, L)], gidx, mask=msk)
                cnt = jnp.sum(msk.astype(jnp.int32))
                return jnp.minimum(off2 + cnt, CAP_T)

            return lax.fori_loop(0, WA // L, ibody, off)

        off = lax.fori_loop(0, NWIN, wbody, 0)
        kbuf[pl.ds(off, L)] = jnp.full((L,), SENT_I32, jnp.int32)
        ibuf[pl.ds(off, L)] = jnp.zeros((L,), jnp.int32)
        r16 = (off + 15) & ~15
        pos = jnp.int32(0)
        for pbit in range(11, -1, -1):
            sz = 16 << pbit
            take = (r16 & sz) != 0
            cur = pos

            @pl.when(take)
            def _(cur=cur, sz=sz):
                pltpu.sync_copy(
                    kbuf.at[pl.ds(pl.multiple_of(cur, 16), sz)], ck_hbm.at[pl.ds(pl.multiple_of(base_w + cur, 16), sz)]
                )
                pltpu.sync_copy(
                    ibuf.at[pl.ds(pl.multiple_of(cur, 16), sz)], ci_hbm.at[pl.ds(pl.multiple_of(base_w + cur, 16), sz)]
                )

            pos = pos + jnp.where(take, sz, 0)

        @pl.when(wid == NW - 1)
        def _():
            def sb(i, c):
                sbuf[pl.ds(i * L, L)] = jnp.full((L,), SENT_I32, jnp.int32)
                return c

            lax.fori_loop(0, WA // L, sb, 0)
            tot = pbuf[pl.ds(24, L)][9]
            nfull = (CANDCAP - tot) // WA

            def fb(i, c):
                pltpu.sync_copy(sbuf, ck_hbm.at[pl.ds(pl.multiple_of(tot + i * WA, 16), WA)])
                return c

            lax.fori_loop(0, nfull, fb, 0)
            rem_base = tot + nfull * WA
            rem = CANDCAP - rem_base
            tpos = rem_base
            for pbit in range(8, -1, -1):
                sz = 16 << pbit
                take = (rem & sz) != 0
                cur = tpos

                @pl.when(take)
                def _(cur=cur, sz=sz):
                    pltpu.sync_copy(sbuf.at[pl.ds(0, sz)], ck_hbm.at[pl.ds(pl.multiple_of(cur, 16), sz)])

                tpos = tpos + jnp.where(take, sz, 0)

    def make_count(shift):
        @functools.partial(
            pl.kernel,
            out_type=jax.ShapeDtypeStruct((NW, BINS_R), jnp.int32),
            mesh=mesh,
            compiler_params=_CP,
            scratch_types=[
                pltpu.VMEM((CHUNK_R,), jnp.int32),
                pltpu.VMEM((L * BINS_R,), jnp.int32),
            ],
        )
        def count_kernel(ck_hbm, hist_hbm, kchunk, histv):
            wid = _wid()
            lane = lax.iota(jnp.int32, L)
            ones = jnp.ones((L,), jnp.int32)

            def zb(i, c):
                histv[pl.ds(i * L, L)] = jnp.zeros((L,), jnp.int32)
                return c

            lax.fori_loop(0, (L * BINS_R) // L, zb, 0)
            pltpu.sync_copy(ck_hbm.at[pl.ds(pl.multiple_of(wid * CHUNK_R, 16), CHUNK_R)], kchunk)

            def ib(j, c):
                x = lax.bitcast_convert_type(kchunk[pl.ds(j * L, L)], jnp.uint32)
                d = ((x >> shift) & jnp.uint32(0xFF)).astype(jnp.int32)
                plsc.addupdate_scatter(histv, [lane * BINS_R + d], ones)
                return c

            lax.fori_loop(0, CHUNK_R // L, ib, 0)

            def mb(c, carry):
                acc = jnp.zeros((L,), jnp.int32)
                for l in range(L):
                    acc = acc + histv[pl.ds(l * BINS_R + c * L, L)]
                histv[pl.ds(c * L, L)] = acc
                return carry

            lax.fori_loop(0, BINS_R // L, mb, 0)
            pltpu.sync_copy(histv.at[pl.ds(0, BINS_R)], hist_hbm.at[wid])

        return count_kernel

    def make_scatter(shift, last):
        if last:
            out_type = (
                jax.ShapeDtypeStruct((CANDCAP,), jnp.float32),
                jax.ShapeDtypeStruct((CANDCAP,), jnp.int32),
            )
        else:
            out_type = (
                jax.ShapeDtypeStruct((CANDCAP,), jnp.int32),
                jax.ShapeDtypeStruct((CANDCAP,), jnp.int32),
            )
        scratch = [
            pltpu.VMEM((CHUNK_R,), jnp.int32),
            pltpu.VMEM((CHUNK_R,), jnp.int32),
            pltpu.VMEM((CHUNK_R // 128, 128), jnp.int32),
            pltpu.VMEM((BINS_R,), jnp.int32),
            pltpu.VMEM((CHUNK_R,), jnp.float32),
            pltpu.SemaphoreType.DMA,
        ]

        @functools.partial(
            pl.kernel, out_type=out_type, mesh=mesh, compiler_params=_CP, scratch_types=scratch
        )
        def scatter_kernel(ck_hbm, ci_hbm, offs_hbm, cko_hbm, cio_hbm,
                           kchunk, ichunk, posb, ctr, vbuf, sem):
            wid = _wid()
            pltpu.sync_copy(ck_hbm.at[pl.ds(pl.multiple_of(wid * CHUNK_R, 16), CHUNK_R)], kchunk)
            pltpu.sync_copy(ci_hbm.at[pl.ds(pl.multiple_of(wid * CHUNK_R, 16), CHUNK_R)], ichunk)
            pltpu.sync_copy(offs_hbm.at[wid], ctr)
            cprobe, _ = plsc.scan_count(jnp.zeros((L,), jnp.int32))
            b0 = jnp.min(cprobe)

            def ib(j, c):
                xu = lax.bitcast_convert_type(kchunk[pl.ds(j * L, L)], jnp.uint32)
                d = ((xu >> shift) & jnp.uint32(0xFF)).astype(jnp.int32)
                cnt, lastm = plsc.scan_count(d)
                rank = cnt - b0
                basev = plsc.load_gather(ctr, [d])
                pos = basev + rank
                plsc.addupdate_scatter(ctr, [d], rank + 1, mask=lastm)
                posb[j // 8, pl.ds((j % 8) * L, L)] = pos
                return c

            lax.fori_loop(0, CHUNK_R // L, ib, 0)

            if last:
                def cb(j, c):
                    key = ~lax.bitcast_convert_type(kchunk[pl.ds(j * L, L)], jnp.uint32)
                    m = key >> 31
                    u = key ^ (jnp.uint32(0xFFFFFFFF) - m * jnp.uint32(0x7FFFFFFF))
                    vbuf[pl.ds(j * L, L)] = lax.bitcast_convert_type(u, jnp.float32)
                    return c

                lax.fori_loop(0, CHUNK_R // L, cb, 0)

            src = vbuf if last else kchunk

            def sbody(r, c):
                pltpu.async_copy(
                    src.at[pl.ds(r * 128, 128)], cko_hbm.at[posb.at[r]], sem
                )
                pltpu.async_copy(
                    ichunk.at[pl.ds(r * 128, 128)], cio_hbm.at[posb.at[r]], sem
                )
                return c

            lax.fori_loop(0, CHUNK_R // 128, sbody, 0)
            pltpu.make_async_copy(ck_hbm.at[pl.ds(0, CHUNK_R)], kchunk, sem).wait()
            pltpu.make_async_copy(ci_hbm.at[pl.ds(0, CHUNK_R)], ichunk, sem).wait()

        return scatter_kernel

    count_kernels = [make_count(8 * p) for p in range(4)]
    scatter_kernels = [make_scatter(8 * p, p == 3) for p in range(4)]
    return hist_kernel, compact_kernel, count_kernels, scatter_kernels


def _pass_offsets(hist):
    h = hist.T.reshape(-1)
    offs = jnp.concatenate(
        [jnp.zeros((1,), jnp.int32), jnp.cumsum(h)[:-1].astype(jnp.int32)]
    )
    return offs.reshape(BINS_R, NW).T


def kernel(weight, k):
    hist_kernel, compact_kernel, count_kernels, scatter_kernels = _build()
    hist = hist_kernel(weight)
    h3 = hist.reshape(NW, L, BINS_A)
    gbin = h3.sum(axis=(0, 1))
    suffix = jnp.cumsum(gbin[::-1])[::-1]
    t1 = jnp.sum((suffix >= KTOP).astype(jnp.int32)) - 1
    thr_u = t1.astype(jnp.uint32) << SHIFT_A
    thr_i = lax.bitcast_convert_type(thr_u, jnp.int32)
    per_tile = h3.sum(axis=1)
    sel = (jnp.arange(BINS_A, dtype=jnp.int32) >= t1)[None, :]
    cnt = jnp.where(sel, per_tile, 0).sum(axis=1).astype(jnp.int32)
    r16 = (cnt + 15) // 16 * 16
    base = jnp.concatenate(
        [jnp.zeros((1,), jnp.int32), jnp.cumsum(r16)[:-1].astype(jnp.int32)]
    )
    total16 = jnp.sum(r16).astype(jnp.int32)
    par = (
        jnp.zeros((40,), jnp.int32)
        .at[0].set(thr_i)
        .at[1:33].set(base)
        .at[33].set(total16)
    )
    ck, ci = compact_kernel(weight, par)
    for p in range(4):
        histp = count_kernels[p](ck)
        offs = _pass_offsets(histp)
        ck, ci = scatter_kernels[p](ck, ci, offs)
    values = ck[:KTOP]
    indices = ci[:KTOP] + (jnp.asarray(k, jnp.int32) - KTOP)
    return values, indices


# speedup vs baseline: 4.6402x; 4.6402x over previous
import functools

import jax
import jax.numpy as jnp
from jax import lax
from jax.experimental import pallas as pl
from jax.experimental.pallas import tpu as pltpu
from jax.experimental.pallas import tpu_sc as plsc

N = 4194304
KTOP = 419431
NC = 2
NS = 16
NW = NC * NS
L = 16

CHUNK_A = N // NW
WA = 8192
NWIN = CHUNK_A // WA
BINS_A = 4096
SHIFT_A = 20

CANDCAP = 589824
CHUNK_R = CANDCAP // NW
CHUNK_S = CANDCAP // NS
PASSES = ((0, 2047), (11, 2047), (22, 1023))
CAP_T = 32768
SENT_I32 = -1


_CP = pltpu.CompilerParams(needs_layout_passes=False)


def _wid():
    return lax.axis_index("s") * NC + lax.axis_index("c")


def _key_from_f32(x):
    u = lax.bitcast_convert_type(x, jnp.uint32)
    m = u >> 31
    return u ^ (m * jnp.uint32(0x7FFFFFFF) + jnp.uint32(0x80000000))


@functools.cache
def _build():
    mesh = plsc.VectorSubcoreMesh(
        core_axis_name="c", subcore_axis_name="s", num_cores=NC, num_subcores=NS
    )

    @functools.partial(
        pl.kernel,
        out_type=jax.ShapeDtypeStruct((NW, L * BINS_A), jnp.int32),
        mesh=mesh,
        compiler_params=_CP,
        scratch_types=[
            pltpu.VMEM((WA,), jnp.float32),
            pltpu.VMEM((L * BINS_A,), jnp.int32),
        ],
    )
    def hist_kernel(w_hbm, hist_hbm, wbuf, hist_v):
        wid = _wid()
        lane = lax.iota(jnp.int32, L)
        ones = jnp.ones((L,), jnp.int32)

        def zbody(i, c):
            hist_v[pl.ds(i * L, L)] = jnp.zeros((L,), jnp.int32)
            return c

        lax.fori_loop(0, (L * BINS_A) // L, zbody, 0)

        base = wid * CHUNK_A

        def wbody(g, c):
            pltpu.sync_copy(w_hbm.at[pl.ds(pl.multiple_of(base + g * WA, 16), WA)], wbuf)

            def ibody(j, c2):
                key = _key_from_f32(wbuf[pl.ds(j * L, L)])
                b = (key >> SHIFT_A).astype(jnp.int32)
                plsc.addupdate_scatter(hist_v, [lane * BINS_A + b], ones)
                return c2

            return lax.fori_loop(0, WA // L, ibody, c)

        lax.fori_loop(0, NWIN, wbody, 0)
        pltpu.sync_copy(hist_v, hist_hbm.at[wid])

    @functools.partial(
        pl.kernel,
        out_type=(
            jax.ShapeDtypeStruct((CANDCAP,), jnp.int32),
            jax.ShapeDtypeStruct((CANDCAP,), jnp.int32),
        ),
        mesh=mesh,
        compiler_params=_CP,
        scratch_types=[
            pltpu.VMEM((WA,), jnp.float32),
            pltpu.VMEM((CAP_T + L,), jnp.int32),
            pltpu.VMEM((CAP_T + L,), jnp.int32),
            pltpu.VMEM((WA,), jnp.int32),
            pltpu.VMEM((40,), jnp.int32),
        ],
    )
    def compact_kernel(w_hbm, par_hbm, ck_hbm, ci_hbm, wbuf, kbuf, ibuf, sbuf, pbuf):
        wid = _wid()
        lane = lax.iota(jnp.int32, L)
        pltpu.sync_copy(par_hbm, pbuf)
        p0 = pbuf[pl.ds(0, L)]
        thr_v = lax.bitcast_convert_type(jnp.full((L,), 0, jnp.int32) + p0[0], jnp.uint32)
        bw_v = plsc.load_gather(pbuf, [jnp.full((L,), 0, jnp.int32) + (1 + wid)])
        base_w = bw_v[0]
        cbase = wid * CHUNK_A

        def wbody(g, off):
            pltpu.sync_copy(w_hbm.at[pl.ds(pl.multiple_of(cbase + g * WA, 16), WA)], wbuf)

            def ibody(j, off2):
                key = _key_from_f32(wbuf[pl.ds(j * L, L)])
                msk = key >= thr_v
                fkey = lax.bitcast_convert_type(~key, jnp.int32)
                gidx = cbase + g * WA + j * L + lane
                plsc.store_compressed(kbuf.at[pl.ds(off2, L)], fkey, mask=msk)
                plsc.store_compressed(ibuf.at[pl.ds(off2, L)], gidx, mask=msk)
                cnt = jnp.sum(msk.astype(jnp.int32))
                return jnp.minimum(off2 + cnt, CAP_T)

            return lax.fori_loop(0, WA // L, ibody, off)

        off = lax.fori_loop(0, NWIN, wbody, 0)
        kbuf[pl.ds(off, L)] = jnp.full((L,), SENT_I32, jnp.int32)
        ibuf[pl.ds(off, L)] = jnp.zeros((L,), jnp.int32)
        r16 = (off + 15) & ~15
        pos = jnp.int32(0)
        for pbit in range(11, -1, -1):
            sz = 16 << pbit
            take = (r16 & sz) != 0
            cur = pos

            @pl.when(take)
            def _(cur=cur, sz=sz):
                pltpu.sync_copy(
                    kbuf.at[pl.ds(pl.multiple_of(cur, 16), sz)], ck_hbm.at[pl.ds(pl.multiple_of(base_w + cur, 16), sz)]
                )
                pltpu.sync_copy(
                    ibuf.at[pl.ds(pl.multiple_of(cur, 16), sz)], ci_hbm.at[pl.ds(pl.multiple_of(base_w + cur, 16), sz)]
                )

            pos = pos + jnp.where(take, sz, 0)

        @pl.when(wid == NW - 1)
        def _():
            def sb(i, c):
                sbuf[pl.ds(i * L, L)] = jnp.full((L,), SENT_I32, jnp.int32)
                return c

            lax.fori_loop(0, WA // L, sb, 0)
            tot = pbuf[pl.ds(24, L)][9]
            nfull = (CANDCAP - tot) // WA

            def fb(i, c):
                pltpu.sync_copy(sbuf, ck_hbm.at[pl.ds(pl.multiple_of(tot + i * WA, 16), WA)])
                return c

            lax.fori_loop(0, nfull, fb, 0)
            rem_base = tot + nfull * WA
            rem = CANDCAP - rem_base
            tpos = rem_base
            for pbit in range(8, -1, -1):
                sz = 16 << pbit
                take = (rem & sz) != 0
                cur = tpos

                @pl.when(take)
                def _(cur=cur, sz=sz):
                    pltpu.sync_copy(sbuf.at[pl.ds(0, sz)], ck_hbm.at[pl.ds(pl.multiple_of(cur, 16), sz)])

                tpos = tpos + jnp.where(take, sz, 0)

    RBINS = 2048

    @functools.partial(
        pl.kernel,
        out_type=jax.ShapeDtypeStruct((NW, RBINS), jnp.int32),
        mesh=mesh,
        compiler_params=_CP,
        scratch_types=[
            pltpu.VMEM((CHUNK_R,), jnp.int32),
            pltpu.VMEM((L * RBINS,), jnp.int32),
            pltpu.VMEM((16,), jnp.int32),
        ],
    )
    def count_kernel(ck_hbm, par_hbm, hist_hbm, kchunk, histv, pbuf):
        wid = _wid()
        lane = lax.iota(jnp.int32, L)
        ones = jnp.ones((L,), jnp.int32)
        pltpu.sync_copy(par_hbm, pbuf)
        pv = pbuf[pl.ds(0, L)]
        shv = (jnp.full((L,), 0, jnp.int32) + pv[0]).astype(jnp.uint32)
        mkv = (jnp.full((L,), 0, jnp.int32) + pv[1]).astype(jnp.uint32)

        def zb(i, c):
            histv[pl.ds(i * L, L)] = jnp.zeros((L,), jnp.int32)
            return c

        lax.fori_loop(0, (L * RBINS) // L, zb, 0)
        pltpu.sync_copy(ck_hbm.at[pl.ds(pl.multiple_of(wid * CHUNK_R, 16), CHUNK_R)], kchunk)

        def ib(j, c):
            x = lax.bitcast_convert_type(kchunk[pl.ds(j * L, L)], jnp.uint32)
            d = ((x >> shv) & mkv).astype(jnp.int32)
            plsc.addupdate_scatter(histv, [lane * RBINS + d], ones)
            return c

        lax.fori_loop(0, CHUNK_R // L, ib, 0)

        def mb(c, carry):
            acc = jnp.zeros((L,), jnp.int32)
            for l in range(L):
                acc = acc + histv[pl.ds(l * RBINS + c * L, L)]
            histv[pl.ds(c * L, L)] = acc
            return carry

        lax.fori_loop(0, RBINS // L, mb, 0)
        pltpu.sync_copy(histv.at[pl.ds(0, RBINS)], hist_hbm.at[wid])

    SEG = 65536
    NRND = 5
    HCH = CHUNK_S // 4

    scatter_out_type = (
        jax.ShapeDtypeStruct((CANDCAP,), jnp.int32),
        jax.ShapeDtypeStruct((CANDCAP,), jnp.int32),
    )
    scatter_scratch = [
        pltpu.VMEM((CHUNK_S,), jnp.int32),
        pltpu.VMEM((CHUNK_S,), jnp.int32),
        pltpu.VMEM((CHUNK_S // 128, 128), jnp.int32),
        pltpu.VMEM((HCH // 128, 128), jnp.int32),
        pltpu.VMEM((RBINS,), jnp.int32),
        pltpu.VMEM((16,), jnp.int32),
        pltpu.VMEM_SHARED((SEG + 32,), jnp.int32),
        pltpu.VMEM_SHARED((SEG + 32,), jnp.int32),
        pltpu.SemaphoreType.DMA,
    ]

    @functools.partial(
        pl.kernel, out_type=scatter_out_type, mesh=mesh, compiler_params=_CP,
        scratch_types=scatter_scratch
    )
    def scatter_kernel(ck_hbm, ci_hbm, par_hbm, offs_hbm, cko_hbm, cio_hbm,
                       kchunk, ichunk, posb, padj, ctr, pbuf, cko_sh, cio_sh, sem):
            sid = lax.axis_index("s")
            cid = lax.axis_index("c")
            cb = sid * CHUNK_S
            pltpu.sync_copy(ck_hbm.at[pl.ds(pl.multiple_of(cb, 16), CHUNK_S)], kchunk)
            pltpu.sync_copy(ci_hbm.at[pl.ds(pl.multiple_of(cb, 16), CHUNK_S)], ichunk)
            pltpu.sync_copy(offs_hbm.at[2 * sid], ctr)
            pltpu.sync_copy(par_hbm, pbuf)
            pv = pbuf[pl.ds(0, L)]
            shv = (jnp.full((L,), 0, jnp.int32) + pv[0]).astype(jnp.uint32)
            mkv = (jnp.full((L,), 0, jnp.int32) + pv[1]).astype(jnp.uint32)
            cprobe, _ = plsc.scan_count(jnp.zeros((L,), jnp.int32))
            b0 = jnp.min(cprobe)

            def ib(j, c):
                xu = lax.bitcast_convert_type(kchunk[pl.ds(j * L, L)], jnp.uint32)
                d = ((xu >> shv) & mkv).astype(jnp.int32)
                cnt, lastm = plsc.scan_count(d)
                rank = cnt - b0
                basev = plsc.load_gather(ctr, [d])
                pos = basev + rank
                plsc.addupdate_scatter(ctr, [d], rank + 1, mask=lastm)
                posb[j // 8, pl.ds((j % 8) * L, L)] = pos
                return c

            lax.fori_loop(0, CHUNK_S // L, ib, 0)

            def round_body(r, carry):
                segbase = (cid * 4 + r) * SEG

                def sub_body(h, c2):
                    def adj(q, c):
                        row = h * (HCH // 128) + q // 8
                        pos = posb[row, pl.ds((q % 8) * L, L)]
                        pos_l = pos - segbase
                        ins = (pos_l >= 0) & (pos_l < SEG)
                        pd = jnp.where(ins, pos_l, SEG + (pos & 31))
                        padj[q // 8, pl.ds((q % 8) * L, L)] = pd
                        return c

                    lax.fori_loop(0, HCH // L, adj, 0)
                    hb = pl.multiple_of(h * HCH, 128)

                    def sb2(t, c):
                        tb = pl.multiple_of(hb + t * 128, 128)
                        pltpu.async_copy(
                            kchunk.at[pl.ds(tb, 128)],
                            cko_sh.at[padj.at[t]], sem,
                        )
                        pltpu.async_copy(
                            ichunk.at[pl.ds(tb, 128)],
                            cio_sh.at[padj.at[t]], sem,
                        )
                        return c

                    lax.fori_loop(0, HCH // 128, sb2, 0)
                    pltpu.make_async_copy(
                        ck_hbm.at[pl.ds(0, HCH)], kchunk.at[pl.ds(0, HCH)], sem
                    ).wait()
                    pltpu.make_async_copy(
                        ck_hbm.at[pl.ds(0, HCH)], ichunk.at[pl.ds(0, HCH)], sem
                    ).wait()
                    return c2

                lax.fori_loop(0, 4, sub_body, 0)
                plsc.subcore_barrier()
                wseg = SEG // NS
                lwb = sid * wseg
                gwb = pl.multiple_of(segbase + lwb, 16)
                pltpu.sync_copy(cko_sh.at[pl.ds(lwb, wseg)],
                                cko_hbm.at[pl.ds(gwb, wseg)])
                pltpu.sync_copy(cio_sh.at[pl.ds(lwb, wseg)],
                                cio_hbm.at[pl.ds(gwb, wseg)])
                plsc.subcore_barrier()
                return carry

            lax.fori_loop(0, NRND, round_body, 0)


    NCONV = 419840

    @functools.partial(
        pl.pallas_call,
        out_shape=jax.ShapeDtypeStruct((NCONV // 1024, 1024), jnp.float32),
    )
    def conv_kernel(fk_ref, out_ref):
        key = ~lax.bitcast_convert_type(fk_ref[...], jnp.uint32)
        m = key >> 31
        u = key ^ (jnp.uint32(0xFFFFFFFF) - m * jnp.uint32(0x7FFFFFFF))
        out_ref[...] = lax.bitcast_convert_type(u, jnp.float32)

    return hist_kernel, compact_kernel, count_kernel, scatter_kernel, conv_kernel


def _pass_offsets(hist):
    totals = hist.sum(axis=0)
    digit_base = jnp.cumsum(totals) - totals
    tile_cum = jnp.cumsum(hist, axis=0) - hist
    return (digit_base[None, :] + tile_cum).astype(jnp.int32)


def kernel(weight, k):
    (hist_kernel, compact_kernel, count_kernel, scatter_kernel,
     conv_kernel) = _build()
    hist = hist_kernel(weight)
    h3 = hist.reshape(NW, L, BINS_A)
    gbin = h3.sum(axis=(0, 1))
    suffix = jnp.cumsum(gbin[::-1])[::-1]
    t1 = jnp.sum((suffix >= KTOP).astype(jnp.int32)) - 1
    thr_u = t1.astype(jnp.uint32) << SHIFT_A
    thr_i = lax.bitcast_convert_type(thr_u, jnp.int32)
    per_tile = h3.sum(axis=1)
    sel = (jnp.arange(BINS_A, dtype=jnp.int32) >= t1)[None, :]
    cnt = jnp.where(sel, per_tile, 0).sum(axis=1).astype(jnp.int32)
    r16 = (cnt + 15) // 16 * 16
    base = jnp.concatenate(
        [jnp.zeros((1,), jnp.int32), jnp.cumsum(r16)[:-1].astype(jnp.int32)]
    )
    total16 = jnp.sum(r16).astype(jnp.int32)
    par = (
        jnp.zeros((40,), jnp.int32)
        .at[0].set(thr_i)
        .at[1:33].set(base)
        .at[33].set(total16)
    )
    ck, ci = compact_kernel(weight, par)
    for shift, dmask in PASSES:
        ppar = jnp.full((16,), 0, jnp.int32).at[0].set(shift).at[1].set(dmask)
        histp = count_kernel(ck, ppar)
        offs = _pass_offsets(histp)
        ck, ci = scatter_kernel(ck, ci, ppar, offs)
    NCONV = 419840
    vals = conv_kernel(ck[:NCONV].reshape(NCONV // 1024, 1024))
    values = vals.reshape(-1)[:KTOP]
    indices = ci[:KTOP] + (jnp.asarray(k, jnp.int32) - KTOP)
    return values, indices


# speedup vs baseline: 4.6918x; 1.0111x over previous
import functools

import jax
import jax.numpy as jnp
from jax import lax
from jax.experimental import pallas as pl
from jax.experimental.pallas import tpu as pltpu
from jax.experimental.pallas import tpu_sc as plsc

N = 4194304
KTOP = 419431
NC = 2
NS = 16
NW = NC * NS
L = 16

CHUNK_A = N // NW
WA = 8192
NWIN = CHUNK_A // WA
BINS_A = 4096
SHIFT_A = 20

CANDCAP = 589824
CHUNK_R = CANDCAP // NW
CHUNK_S = CANDCAP // NS
PASSES = ((0, 2047), (11, 2047), (22, 1023))
CAP_T = 32768
SENT_I32 = -1


_CP = pltpu.CompilerParams(needs_layout_passes=False)


def _wid():
    return lax.axis_index("s") * NC + lax.axis_index("c")


def _key_from_f32(x):
    u = lax.bitcast_convert_type(x, jnp.uint32)
    m = u >> 31
    return u ^ (m * jnp.uint32(0x7FFFFFFF) + jnp.uint32(0x80000000))


@functools.cache
def _build():
    mesh = plsc.VectorSubcoreMesh(
        core_axis_name="c", subcore_axis_name="s", num_cores=NC, num_subcores=NS
    )

    @functools.partial(
        pl.kernel,
        out_type=jax.ShapeDtypeStruct((NW, L * BINS_A), jnp.int32),
        mesh=mesh,
        compiler_params=_CP,
        scratch_types=[
            pltpu.VMEM((2, WA), jnp.float32),
            pltpu.VMEM((L * BINS_A,), jnp.int32),
            pltpu.SemaphoreType.DMA,
        ],
    )
    def hist_kernel(w_hbm, hist_hbm, wbuf, hist_v, dsem):
        wid = _wid()
        lane = lax.iota(jnp.int32, L)
        ones = jnp.ones((L,), jnp.int32)

        def zbody(i, c):
            hist_v[pl.ds(i * L, L)] = jnp.zeros((L,), jnp.int32)
            return c

        lax.fori_loop(0, (L * BINS_A) // L, zbody, 0)

        base = wid * CHUNK_A
        pltpu.async_copy(
            w_hbm.at[pl.ds(pl.multiple_of(base, 16), WA)], wbuf.at[0], dsem
        )

        def wbody(g, c):
            pltpu.make_async_copy(
                w_hbm.at[pl.ds(0, WA)], wbuf.at[0], dsem
            ).wait()

            @pl.when(g + 1 < NWIN)
            def _():
                pltpu.async_copy(
                    w_hbm.at[pl.ds(pl.multiple_of(base + (g + 1) * WA, 16), WA)],
                    wbuf.at[(g + 1) % 2], dsem,
                )

            def ibody(j, c2):
                key = _key_from_f32(wbuf[g % 2, pl.ds(j * L, L)])
                b = (key >> SHIFT_A).astype(jnp.int32)
                plsc.addupdate_scatter(hist_v, [lane * BINS_A + b], ones)
                return c2

            return lax.fori_loop(0, WA // L, ibody, c)

        lax.fori_loop(0, NWIN, wbody, 0)
        pltpu.sync_copy(hist_v, hist_hbm.at[wid])

    @functools.partial(
        pl.kernel,
        out_type=(
            jax.ShapeDtypeStruct((CANDCAP,), jnp.int32),
            jax.ShapeDtypeStruct((CANDCAP,), jnp.int32),
        ),
        mesh=mesh,
        compiler_params=_CP,
        scratch_types=[
            pltpu.VMEM((2, WA), jnp.float32),
            pltpu.VMEM((CAP_T + L,), jnp.int32),
            pltpu.VMEM((CAP_T + L,), jnp.int32),
            pltpu.VMEM((WA,), jnp.int32),
            pltpu.VMEM((40,), jnp.int32),
            pltpu.SemaphoreType.DMA,
        ],
    )
    def compact_kernel(w_hbm, par_hbm, ck_hbm, ci_hbm, wbuf, kbuf, ibuf, sbuf,
                       pbuf, dsem):
        wid = _wid()
        lane = lax.iota(jnp.int32, L)
        pltpu.sync_copy(par_hbm, pbuf)
        p0 = pbuf[pl.ds(0, L)]
        thr_v = lax.bitcast_convert_type(jnp.full((L,), 0, jnp.int32) + p0[0], jnp.uint32)
        bw_v = plsc.load_gather(pbuf, [jnp.full((L,), 0, jnp.int32) + (1 + wid)])
        base_w = bw_v[0]
        cbase = wid * CHUNK_A
        pltpu.async_copy(
            w_hbm.at[pl.ds(pl.multiple_of(cbase, 16), WA)], wbuf.at[0], dsem
        )

        def wbody(g, off):
            pltpu.make_async_copy(
                w_hbm.at[pl.ds(0, WA)], wbuf.at[0], dsem
            ).wait()

            @pl.when(g + 1 < NWIN)
            def _():
                pltpu.async_copy(
                    w_hbm.at[pl.ds(pl.multiple_of(cbase + (g + 1) * WA, 16), WA)],
                    wbuf.at[(g + 1) % 2], dsem,
                )

            def ibody(j, off2):
                key = _key_from_f32(wbuf[g % 2, pl.ds(j * L, L)])
                msk = key >= thr_v
                fkey = lax.bitcast_convert_type(~key, jnp.int32)
                gidx = cbase + g * WA + j * L + lane
                plsc.store_compressed(kbuf.at[pl.ds(off2, L)], fkey, mask=msk)
                plsc.store_compressed(ibuf.at[pl.ds(off2, L)], gidx, mask=msk)
                cnt = jnp.sum(msk.astype(jnp.int32))
                return jnp.minimum(off2 + cnt, CAP_T)

            return lax.fori_loop(0, WA // L, ibody, off)

        off = lax.fori_loop(0, NWIN, wbody, 0)
        kbuf[pl.ds(off, L)] = jnp.full((L,), SENT_I32, jnp.int32)
        ibuf[pl.ds(off, L)] = jnp.zeros((L,), jnp.int32)
        r16 = (off + 15) & ~15
        pos = jnp.int32(0)
        for pbit in range(11, -1, -1):
            sz = 16 << pbit
            take = (r16 & sz) != 0
            cur = pos

            @pl.when(take)
            def _(cur=cur, sz=sz):
                pltpu.sync_copy(
                    kbuf.at[pl.ds(pl.multiple_of(cur, 16), sz)], ck_hbm.at[pl.ds(pl.multiple_of(base_w + cur, 16), sz)]
                )
                pltpu.sync_copy(
                    ibuf.at[pl.ds(pl.multiple_of(cur, 16), sz)], ci_hbm.at[pl.ds(pl.multiple_of(base_w + cur, 16), sz)]
                )

            pos = pos + jnp.where(take, sz, 0)

        @pl.when(wid == NW - 1)
        def _():
            def sb(i, c):
                sbuf[pl.ds(i * L, L)] = jnp.full((L,), SENT_I32, jnp.int32)
                return c

            lax.fori_loop(0, WA // L, sb, 0)
            tot = pbuf[pl.ds(24, L)][9]
            nfull = (CANDCAP - tot) // WA

            def fb(i, c):
                pltpu.sync_copy(sbuf, ck_hbm.at[pl.ds(pl.multiple_of(tot + i * WA, 16), WA)])
                return c

            lax.fori_loop(0, nfull, fb, 0)
            rem_base = tot + nfull * WA
            rem = CANDCAP - rem_base
            tpos = rem_base
            for pbit in range(8, -1, -1):
                sz = 16 << pbit
                take = (rem & sz) != 0
                cur = tpos

                @pl.when(take)
                def _(cur=cur, sz=sz):
                    pltpu.sync_copy(sbuf.at[pl.ds(0, sz)], ck_hbm.at[pl.ds(pl.multiple_of(cur, 16), sz)])

                tpos = tpos + jnp.where(take, sz, 0)

    RBINS = 2048

    @functools.partial(
        pl.kernel,
        out_type=jax.ShapeDtypeStruct((NW, RBINS), jnp.int32),
        mesh=mesh,
        compiler_params=_CP,
        scratch_types=[
            pltpu.VMEM((CHUNK_R,), jnp.int32),
            pltpu.VMEM((L * RBINS,), jnp.int32),
            pltpu.VMEM((16,), jnp.int32),
        ],
    )
    def count_kernel(ck_hbm, par_hbm, hist_hbm, kchunk, histv, pbuf):
        wid = _wid()
        lane = lax.iota(jnp.int32, L)
        ones = jnp.ones((L,), jnp.int32)
        pltpu.sync_copy(par_hbm, pbuf)
        pv = pbuf[pl.ds(0, L)]
        shv = (jnp.full((L,), 0, jnp.int32) + pv[0]).astype(jnp.uint32)
        mkv = (jnp.full((L,), 0, jnp.int32) + pv[1]).astype(jnp.uint32)

        def zb(i, c):
            histv[pl.ds(i * L, L)] = jnp.zeros((L,), jnp.int32)
            return c

        lax.fori_loop(0, (L * RBINS) // L, zb, 0)
        pltpu.sync_copy(ck_hbm.at[pl.ds(pl.multiple_of(wid * CHUNK_R, 16), CHUNK_R)], kchunk)

        def ib(j, c):
            x = lax.bitcast_convert_type(kchunk[pl.ds(j * L, L)], jnp.uint32)
            d = ((x >> shv) & mkv).astype(jnp.int32)
            plsc.addupdate_scatter(histv, [lane * RBINS + d], ones)
            return c

        lax.fori_loop(0, CHUNK_R // L, ib, 0)

        def mb(c, carry):
            acc = jnp.zeros((L,), jnp.int32)
            for l in range(L):
                acc = acc + histv[pl.ds(l * RBINS + c * L, L)]
            histv[pl.ds(c * L, L)] = acc
            return carry

        lax.fori_loop(0, RBINS // L, mb, 0)
        pltpu.sync_copy(histv.at[pl.ds(0, RBINS)], hist_hbm.at[wid])

    SEG = 65536
    NRND = 5
    HCH = CHUNK_S // 4

    scatter_out_type = (
        jax.ShapeDtypeStruct((CANDCAP,), jnp.int32),
        jax.ShapeDtypeStruct((CANDCAP,), jnp.int32),
    )
    scatter_scratch = [
        pltpu.VMEM((CHUNK_S,), jnp.int32),
        pltpu.VMEM((CHUNK_S,), jnp.int32),
        pltpu.VMEM((CHUNK_S // 128, 128), jnp.int32),
        pltpu.VMEM((HCH // 128, 128), jnp.int32),
        pltpu.VMEM((RBINS,), jnp.int32),
        pltpu.VMEM((16,), jnp.int32),
        pltpu.VMEM_SHARED((SEG + 32,), jnp.int32),
        pltpu.VMEM_SHARED((SEG + 32,), jnp.int32),
        pltpu.SemaphoreType.DMA,
    ]

    @functools.partial(
        pl.kernel, out_type=scatter_out_type, mesh=mesh, compiler_params=_CP,
        scratch_types=scatter_scratch
    )
    def scatter_kernel(ck_hbm, ci_hbm, par_hbm, offs_hbm, cko_hbm, cio_hbm,
                       kchunk, ichunk, posb, padj, ctr, pbuf, cko_sh, cio_sh, sem):
            sid = lax.axis_index("s")
            cid = lax.axis_index("c")
            cb = sid * CHUNK_S
            pltpu.sync_copy(ck_hbm.at[pl.ds(pl.multiple_of(cb, 16), CHUNK_S)], kchunk)
            pltpu.sync_copy(ci_hbm.at[pl.ds(pl.multiple_of(cb, 16), CHUNK_S)], ichunk)
            pltpu.sync_copy(offs_hbm.at[2 * sid], ctr)
            pltpu.sync_copy(par_hbm, pbuf)
            pv = pbuf[pl.ds(0, L)]
            shv = (jnp.full((L,), 0, jnp.int32) + pv[0]).astype(jnp.uint32)
            mkv = (jnp.full((L,), 0, jnp.int32) + pv[1]).astype(jnp.uint32)
            cprobe, _ = plsc.scan_count(jnp.zeros((L,), jnp.int32))
            b0 = jnp.min(cprobe)

            def ib(j, c):
                xu = lax.bitcast_convert_type(kchunk[pl.ds(j * L, L)], jnp.uint32)
                d = ((xu >> shv) & mkv).astype(jnp.int32)
                cnt, lastm = plsc.scan_count(d)
                rank = cnt - b0
                basev = plsc.load_gather(ctr, [d])
                pos = basev + rank
                plsc.addupdate_scatter(ctr, [d], rank + 1, mask=lastm)
                posb[j // 8, pl.ds((j % 8) * L, L)] = pos
                return c

            lax.fori_loop(0, CHUNK_S // L, ib, 0)

            def round_body(r, carry):
                segbase = (cid * 4 + r) * SEG

                def sub_body(h, c2):
                    def adj(q, c):
                        row = h * (HCH // 128) + q // 8
                        pos = posb[row, pl.ds((q % 8) * L, L)]
                        pos_l = pos - segbase
                        ins = (pos_l >= 0) & (pos_l < SEG)
                        pd = jnp.where(ins, pos_l, SEG + (pos & 31))
                        padj[q // 8, pl.ds((q % 8) * L, L)] = pd
                        return c

                    lax.fori_loop(0, HCH // L, adj, 0)
                    hb = pl.multiple_of(h * HCH, 128)

                    def sb2(t, c):
                        tb = pl.multiple_of(hb + t * 128, 128)
                        pltpu.async_copy(
                            kchunk.at[pl.ds(tb, 128)],
                            cko_sh.at[padj.at[t]], sem,
                        )
                        pltpu.async_copy(
                            ichunk.at[pl.ds(tb, 128)],
                            cio_sh.at[padj.at[t]], sem,
                        )
                        return c

                    lax.fori_loop(0, HCH // 128, sb2, 0)
                    pltpu.make_async_copy(
                        ck_hbm.at[pl.ds(0, HCH)], kchunk.at[pl.ds(0, HCH)], sem
                    ).wait()
                    pltpu.make_async_copy(
                        ck_hbm.at[pl.ds(0, HCH)], ichunk.at[pl.ds(0, HCH)], sem
                    ).wait()
                    return c2

                lax.fori_loop(0, 4, sub_body, 0)
                plsc.subcore_barrier()
                wseg = SEG // NS
                lwb = sid * wseg
                gwb = pl.multiple_of(segbase + lwb, 16)
                pltpu.sync_copy(cko_sh.at[pl.ds(lwb, wseg)],
                                cko_hbm.at[pl.ds(gwb, wseg)])
                pltpu.sync_copy(cio_sh.at[pl.ds(lwb, wseg)],
                                cio_hbm.at[pl.ds(gwb, wseg)])
                plsc.subcore_barrier()
                return carry

            lax.fori_loop(0, NRND, round_body, 0)


    NCONV = 419840

    @functools.partial(
        pl.pallas_call,
        out_shape=jax.ShapeDtypeStruct((NCONV // 1024, 1024), jnp.float32),
    )
    def conv_kernel(fk_ref, out_ref):
        key = ~lax.bitcast_convert_type(fk_ref[...], jnp.uint32)
        m = key >> 31
        u = key ^ (jnp.uint32(0xFFFFFFFF) - m * jnp.uint32(0x7FFFFFFF))
        out_ref[...] = lax.bitcast_convert_type(u, jnp.float32)

    return hist_kernel, compact_kernel, count_kernel, scatter_kernel, conv_kernel


def _pass_offsets(hist):
    totals = hist.sum(axis=0)
    digit_base = jnp.cumsum(totals) - totals
    tile_cum = jnp.cumsum(hist, axis=0) - hist
    return (digit_base[None, :] + tile_cum).astype(jnp.int32)


def kernel(weight, k):
    (hist_kernel, compact_kernel, count_kernel, scatter_kernel,
     conv_kernel) = _build()
    hist = hist_kernel(weight)
    h3 = hist.reshape(NW, L, BINS_A)
    gbin = h3.sum(axis=(0, 1))
    suffix = jnp.cumsum(gbin[::-1])[::-1]
    t1 = jnp.sum((suffix >= KTOP).astype(jnp.int32)) - 1
    thr_u = t1.astype(jnp.uint32) << SHIFT_A
    thr_i = lax.bitcast_convert_type(thr_u, jnp.int32)
    per_tile = h3.sum(axis=1)
    sel = (jnp.arange(BINS_A, dtype=jnp.int32) >= t1)[None, :]
    cnt = jnp.where(sel, per_tile, 0).sum(axis=1).astype(jnp.int32)
    r16 = (cnt + 15) // 16 * 16
    base = jnp.concatenate(
        [jnp.zeros((1,), jnp.int32), jnp.cumsum(r16)[:-1].astype(jnp.int32)]
    )
    total16 = jnp.sum(r16).astype(jnp.int32)
    par = (
        jnp.zeros((40,), jnp.int32)
        .at[0].set(thr_i)
        .at[1:33].set(base)
        .at[33].set(total16)
    )
    ck, ci = compact_kernel(weight, par)
    for shift, dmask in PASSES:
        ppar = jnp.full((16,), 0, jnp.int32).at[0].set(shift).at[1].set(dmask)
        histp = count_kernel(ck, ppar)
        offs = _pass_offsets(histp)
        ck, ci = scatter_kernel(ck, ci, ppar, offs)
    NCONV = 419840
    vals = conv_kernel(ck[:NCONV].reshape(NCONV // 1024, 1024))
    values = vals.reshape(-1)[:KTOP]
    indices = ci[:KTOP] + (jnp.asarray(k, jnp.int32) - KTOP)
    return values, indices
